# Initial kernel scaffold; baseline (speedup 1.0000x reference)
import jax
import jax.numpy as jnp
from jax.experimental import pallas as pl


def kernel(anchors, deltas, scores):
    raise NotImplementedError("write your pallas kernel here")


# speedup vs baseline: 135.2646x; 135.2646x over previous
import functools

import jax
import jax.numpy as jnp
from jax import lax
from jax.experimental import pallas as pl
from jax.experimental.pallas import tpu as pltpu

_IOU_T = 0.5
_WIN = (0.0, 0.0, 512.0, 512.0)
_L = 128
_SENT = -1.0e6


def _nms_body(sa_ref, sd_ref, boxes_ref, keep_ref,
              y1r, x1r, y2r, x2r, arr,
              sy1, sx1, sy2, sx2, sar, *, nrows, nsrv):
    a0, a1, a2, a3 = sa_ref[0], sa_ref[1], sa_ref[2], sa_ref[3]
    d0, d1, d2, d3 = sd_ref[0], sd_ref[1], sd_ref[2], sd_ref[3]
    height = a2 - a0
    width = a3 - a1
    cy = a0 + 0.5 * height
    cx = a1 + 0.5 * width
    cy = cy + d0 * height
    cx = cx + d1 * width
    height = height * jnp.exp(d2)
    width = width * jnp.exp(d3)
    y1 = cy - 0.5 * height
    x1 = cx - 0.5 * width
    y2 = y1 + height
    x2 = x1 + width
    y1 = jnp.clip(y1, _WIN[0], _WIN[2])
    x1 = jnp.clip(x1, _WIN[1], _WIN[3])
    y2 = jnp.clip(y2, _WIN[0], _WIN[2])
    x2 = jnp.clip(x2, _WIN[1], _WIN[3])
    boxes_ref[0] = y1
    boxes_ref[1] = x1
    boxes_ref[2] = y2
    boxes_ref[3] = x2
    y1r[...] = y1
    x1r[...] = x1
    y2r[...] = y2
    x2r[...] = x2
    arr[...] = (y2 - y1 + 1.0) * (x2 - x1 + 1.0)

    lane = lax.broadcasted_iota(jnp.int32, (1, _L), 1)
    rr_i = lax.broadcasted_iota(jnp.int32, (_L, _L), 0)
    cc_i = lax.broadcasted_iota(jnp.int32, (_L, _L), 1)
    lower = rr_i > cc_i
    lte = (rr_i <= cc_i).astype(jnp.float32)

    sent_row = jnp.full((1, _L), _SENT, jnp.float32)
    one_row = jnp.ones((1, _L), jnp.float32)

    sy1[...] = jnp.full((nsrv, _L), _SENT, jnp.float32)
    sx1[...] = jnp.full((nsrv, _L), _SENT, jnp.float32)
    sy2[...] = jnp.full((nsrv, _L), _SENT, jnp.float32)
    sx2[...] = jnp.full((nsrv, _L), _SENT, jnp.float32)
    sar[...] = jnp.ones((nsrv, _L), jnp.float32)

    def block_step(b, carry):
        nfull, c, gy1, gx1, gy2, gx2, gar = carry

        sy1[pl.ds(nfull, 1), :] = gy1
        sx1[pl.ds(nfull, 1), :] = gx1
        sy2[pl.ds(nfull, 1), :] = gy2
        sx2[pl.ds(nfull, 1), :] = gx2
        sar[pl.ds(nfull, 1), :] = gar

        by1 = y1r[pl.ds(b, 1), :]
        bx1 = x1r[pl.ds(b, 1), :]
        by2 = y2r[pl.ds(b, 1), :]
        bx2 = x2r[pl.ds(b, 1), :]
        bar = arr[pl.ds(b, 1), :]
        by1c = jnp.transpose(by1)
        bx1c = jnp.transpose(bx1)
        by2c = jnp.transpose(by2)
        bx2c = jnp.transpose(bx2)
        barc = jnp.transpose(bar)

        def cross(t, acc):
            gy1t = sy1[pl.ds(8 * t, 8), :]
            gx1t = sx1[pl.ds(8 * t, 8), :]
            gy2t = sy2[pl.ds(8 * t, 8), :]
            gx2t = sx2[pl.ds(8 * t, 8), :]
            gart = sar[pl.ds(8 * t, 8), :]
            for u in range(8):
                ry1 = gy1t[u:u + 1, :]
                rx1 = gx1t[u:u + 1, :]
                ry2 = gy2t[u:u + 1, :]
                rx2 = gx2t[u:u + 1, :]
                rar = gart[u:u + 1, :]
                cy1 = jnp.maximum(by1c, ry1)
                cx1 = jnp.maximum(bx1c, rx1)
                cy2 = jnp.minimum(by2c, ry2)
                cx2 = jnp.minimum(bx2c, rx2)
                ch = jnp.maximum(0.0, cy2 - cy1 + 1.0)
                cw = jnp.maximum(0.0, cx2 - cx1 + 1.0)
                cinter = ch * cw
                ciou = cinter / (barc + rar - cinter)
                acc = acc | jnp.where(ciou >= _IOU_T, 1, 0)
            return acc

        trips = (nfull >> 3) + 1
        acc_mat = lax.fori_loop(0, trips, cross,
                                jnp.zeros((_L, _L), jnp.int32))
        s0 = jnp.max(acc_mat, axis=1, keepdims=True)

        yy1 = jnp.maximum(by1c, by1)
        xx1 = jnp.maximum(bx1c, bx1)
        yy2 = jnp.minimum(by2c, by2)
        xx2 = jnp.minimum(bx2c, bx2)
        hgt = jnp.maximum(0.0, yy2 - yy1 + 1.0)
        wdt = jnp.maximum(0.0, xx2 - xx1 + 1.0)
        inter = hgt * wdt
        iou = inter / (barc + bar - inter)
        mt = jnp.where((iou >= _IOU_T) & lower, 1, 0)

        def fp_cond(cr):
            s, sp, it = cr
            return (it < 1) | (jnp.max(jnp.abs(s - sp)) > 0)

        def fp_body(cr):
            s, sp, it = cr
            kept_row = jnp.where(jnp.transpose(s) == 0, 1, 0)
            s_new = s0 | jnp.max(mt * kept_row, axis=1, keepdims=True)
            return (s_new, s, it + 1)

        s, _, _ = lax.while_loop(fp_cond, fp_body, (s0, s0, 0))

        srow = jnp.transpose(s)
        keep_ref[pl.ds(b, 1), :] = srow == 0

        kf = jnp.where(srow == 0, 1.0, 0.0)
        cs = lax.dot_general(kf, lte, (((1,), (0,)), ((), ())),
                             precision=lax.Precision.HIGHEST,
                             preferred_element_type=jnp.float32)
        k = jnp.sum(kf).astype(jnp.int32)
        csc = jnp.transpose(cs)
        kfc = jnp.transpose(kf)
        pt = jnp.where((csc - 1.0 == lane.astype(jnp.float32)) & (kfc > 0.0),
                       1.0, 0.0)
        stack = jnp.concatenate([by1, bx1, by2, bx2, bar,
                                 sent_row, sent_row, sent_row], axis=0)
        comp = lax.dot_general(stack, pt, (((1,), (0,)), ((), ())),
                               precision=lax.Precision.HIGHEST,
                               preferred_element_type=jnp.float32)
        valid = lane < k
        cmy1 = jnp.where(valid, comp[0:1, :], sent_row)
        cmx1 = jnp.where(valid, comp[1:2, :], sent_row)
        cmy2 = jnp.where(valid, comp[2:3, :], sent_row)
        cmx2 = jnp.where(valid, comp[3:4, :], sent_row)
        cmar = jnp.where(valid, comp[4:5, :], one_row)

        rly1 = pltpu.roll(cmy1, c, 1)
        rlx1 = pltpu.roll(cmx1, c, 1)
        rly2 = pltpu.roll(cmy2, c, 1)
        rlx2 = pltpu.roll(cmx2, c, 1)
        rlar = pltpu.roll(cmar, c, 1)
        in_hi = lane >= c
        mgy1 = jnp.where(in_hi, rly1, gy1)
        mgx1 = jnp.where(in_hi, rlx1, gx1)
        mgy2 = jnp.where(in_hi, rly2, gy2)
        mgx2 = jnp.where(in_hi, rlx2, gx2)
        mgar = jnp.where(in_hi, rlar, gar)

        flush = (c + k) >= _L
        c_new = jnp.where(flush, c + k - _L, c + k)

        @pl.when(flush)
        def _():
            sy1[pl.ds(nfull, 1), :] = mgy1
            sx1[pl.ds(nfull, 1), :] = mgx1
            sy2[pl.ds(nfull, 1), :] = mgy2
            sx2[pl.ds(nfull, 1), :] = mgx2
            sar[pl.ds(nfull, 1), :] = mgar

        nfull_new = nfull + jnp.where(flush, 1, 0)
        in_lo = lane < c_new
        ny1 = jnp.where(flush, jnp.where(in_lo, rly1, sent_row), mgy1)
        nx1 = jnp.where(flush, jnp.where(in_lo, rlx1, sent_row), mgx1)
        ny2 = jnp.where(flush, jnp.where(in_lo, rly2, sent_row), mgy2)
        nx2 = jnp.where(flush, jnp.where(in_lo, rlx2, sent_row), mgx2)
        nar = jnp.where(flush, jnp.where(in_lo, rlar, one_row), mgar)

        return (nfull_new, c_new, ny1, nx1, ny2, nx2, nar)

    init = (jnp.int32(0), jnp.int32(0), sent_row, sent_row, sent_row,
            sent_row, one_row)
    lax.fori_loop(0, nrows, block_step, init)


def kernel(anchors, deltas, scores):
    n = scores.shape[0]
    nrows = (n + _L - 1) // _L
    pad = nrows * _L - n

    order = jnp.argsort(-scores)
    sa = jnp.pad(anchors[order], ((0, pad), (0, 0)))
    sd = jnp.pad(deltas[order], ((0, pad), (0, 0)))
    ss = scores[order]

    sa4 = sa.T.reshape(4, nrows, _L)
    sd4 = sd.T.reshape(4, nrows, _L)

    nsrv = ((nrows + 8) // 8) * 8
    boxes4, keep = pl.pallas_call(
        functools.partial(_nms_body, nrows=nrows, nsrv=nsrv),
        out_shape=[
            jax.ShapeDtypeStruct((4, nrows, _L), jnp.float32),
            jax.ShapeDtypeStruct((nrows, _L), jnp.bool_),
        ],
        scratch_shapes=(
            [pltpu.VMEM((nrows, _L), jnp.float32)] * 5
            + [pltpu.VMEM((nsrv, _L), jnp.float32)] * 5
        ),
    )(sa4, sd4)

    boxes_s = boxes4.reshape(4, nrows * _L)[:, :n].T
    keep_s = keep.reshape(nrows * _L)[:n]
    dets_sorted = jnp.concatenate([boxes_s, ss[:, None]], axis=1)
    dets_sorted = dets_sorted * keep_s[:, None].astype(jnp.float32)
    return jnp.zeros((n, 5), jnp.float32).at[order].add(dets_sorted)


# speedup vs baseline: 138.7449x; 1.0257x over previous
import functools

import jax
import jax.numpy as jnp
from jax import lax
from jax.experimental import pallas as pl
from jax.experimental.pallas import tpu as pltpu

_IOU_T = 0.5
_WIN = (0.0, 0.0, 512.0, 512.0)
_L = 128
_SENT = -1.0e6


def _nms_body(sa_ref, sd_ref, boxes_ref, keep_ref,
              y1r, x1r, y2r, x2r, arr,
              sy1, sx1, sy2, sx2, sar, accr, *, nrows, nsrv):
    a0, a1, a2, a3 = sa_ref[0], sa_ref[1], sa_ref[2], sa_ref[3]
    d0, d1, d2, d3 = sd_ref[0], sd_ref[1], sd_ref[2], sd_ref[3]
    height = a2 - a0
    width = a3 - a1
    cy = a0 + 0.5 * height
    cx = a1 + 0.5 * width
    cy = cy + d0 * height
    cx = cx + d1 * width
    height = height * jnp.exp(d2)
    width = width * jnp.exp(d3)
    y1 = cy - 0.5 * height
    x1 = cx - 0.5 * width
    y2 = y1 + height
    x2 = x1 + width
    y1 = jnp.clip(y1, _WIN[0], _WIN[2])
    x1 = jnp.clip(x1, _WIN[1], _WIN[3])
    y2 = jnp.clip(y2, _WIN[0], _WIN[2])
    x2 = jnp.clip(x2, _WIN[1], _WIN[3])
    boxes_ref[0] = y1
    boxes_ref[1] = x1
    boxes_ref[2] = y2
    boxes_ref[3] = x2
    y1r[...] = y1
    x1r[...] = x1
    y2r[...] = y2
    x2r[...] = x2
    arr[...] = (y2 - y1 + 1.0) * (x2 - x1 + 1.0)

    lane = lax.broadcasted_iota(jnp.int32, (1, _L), 1)
    rr_i = lax.broadcasted_iota(jnp.int32, (_L, _L), 0)
    cc_i = lax.broadcasted_iota(jnp.int32, (_L, _L), 1)
    lower = rr_i > cc_i
    lte = (rr_i <= cc_i).astype(jnp.float32)

    sent_row = jnp.full((1, _L), _SENT, jnp.float32)
    one_row = jnp.ones((1, _L), jnp.float32)

    sy1[...] = jnp.full((nsrv, _L), _SENT, jnp.float32)
    sx1[...] = jnp.full((nsrv, _L), _SENT, jnp.float32)
    sy2[...] = jnp.full((nsrv, _L), _SENT, jnp.float32)
    sx2[...] = jnp.full((nsrv, _L), _SENT, jnp.float32)
    sar[...] = jnp.ones((nsrv, _L), jnp.float32)

    def block_step(b, carry):
        nfull, c, gy1, gx1, gy2, gx2, gar = carry

        sy1[pl.ds(nfull, 1), :] = gy1
        sx1[pl.ds(nfull, 1), :] = gx1
        sy2[pl.ds(nfull, 1), :] = gy2
        sx2[pl.ds(nfull, 1), :] = gx2
        sar[pl.ds(nfull, 1), :] = gar

        by1 = y1r[pl.ds(b, 1), :]
        bx1 = x1r[pl.ds(b, 1), :]
        by2 = y2r[pl.ds(b, 1), :]
        bx2 = x2r[pl.ds(b, 1), :]
        bar = arr[pl.ds(b, 1), :]
        by1c = jnp.transpose(by1)
        bx1c = jnp.transpose(bx1)
        by2c = jnp.transpose(by2)
        bx2c = jnp.transpose(bx2)
        barc = jnp.transpose(bar)

        accr[...] = jnp.zeros((_L, _L), jnp.int32)

        def cross(t, _):
            acc = jnp.zeros((_L, _L), jnp.int32)
            gy1t = sy1[pl.ds(8 * t, 8), :]
            gx1t = sx1[pl.ds(8 * t, 8), :]
            gy2t = sy2[pl.ds(8 * t, 8), :]
            gx2t = sx2[pl.ds(8 * t, 8), :]
            gart = sar[pl.ds(8 * t, 8), :]
            for u in range(8):
                ry1 = gy1t[u:u + 1, :]
                rx1 = gx1t[u:u + 1, :]
                ry2 = gy2t[u:u + 1, :]
                rx2 = gx2t[u:u + 1, :]
                rar = gart[u:u + 1, :]
                cy1 = jnp.maximum(by1c, ry1)
                cx1 = jnp.maximum(bx1c, rx1)
                cy2 = jnp.minimum(by2c, ry2)
                cx2 = jnp.minimum(bx2c, rx2)
                ch = jnp.maximum(0.0, cy2 - cy1 + 1.0)
                cw = jnp.maximum(0.0, cx2 - cx1 + 1.0)
                cinter = ch * cw
                ciou = cinter / (barc + rar - cinter)
                acc = acc | jnp.where(ciou >= _IOU_T, 1, 0)
            accr[...] = accr[...] | acc
            return 0

        trips = (nfull >> 3) + 1
        lax.fori_loop(0, trips, cross, 0)
        s0 = jnp.max(accr[...], axis=1, keepdims=True)

        yy1 = jnp.maximum(by1c, by1)
        xx1 = jnp.maximum(bx1c, bx1)
        yy2 = jnp.minimum(by2c, by2)
        xx2 = jnp.minimum(bx2c, bx2)
        hgt = jnp.maximum(0.0, yy2 - yy1 + 1.0)
        wdt = jnp.maximum(0.0, xx2 - xx1 + 1.0)
        inter = hgt * wdt
        iou = inter / (barc + bar - inter)
        mt = jnp.where((iou >= _IOU_T) & lower, 1, 0)

        def fp_cond(cr):
            s, sp, it = cr
            return (it < 1) | (jnp.max(jnp.abs(s - sp)) > 0)

        def fp_body(cr):
            s, sp, it = cr
            kept_row = jnp.where(jnp.transpose(s) == 0, 1, 0)
            s_new = s0 | jnp.max(mt * kept_row, axis=1, keepdims=True)
            return (s_new, s, it + 1)

        s, _, _ = lax.while_loop(fp_cond, fp_body, (s0, s0, 0))

        srow = jnp.transpose(s)
        keep_ref[pl.ds(b, 1), :] = srow == 0

        kf = jnp.where(srow == 0, 1.0, 0.0)
        cs = lax.dot_general(kf, lte, (((1,), (0,)), ((), ())),
                             precision=lax.Precision.HIGHEST,
                             preferred_element_type=jnp.float32)
        k = jnp.sum(kf).astype(jnp.int32)
        csc = jnp.transpose(cs)
        kfc = jnp.transpose(kf)
        pt = jnp.where((csc - 1.0 == lane.astype(jnp.float32)) & (kfc > 0.0),
                       1.0, 0.0)
        stack = jnp.concatenate([by1, bx1, by2, bx2, bar,
                                 sent_row, sent_row, sent_row], axis=0)
        comp = lax.dot_general(stack, pt, (((1,), (0,)), ((), ())),
                               precision=lax.Precision.HIGHEST,
                               preferred_element_type=jnp.float32)
        valid = lane < k
        cmy1 = jnp.where(valid, comp[0:1, :], sent_row)
        cmx1 = jnp.where(valid, comp[1:2, :], sent_row)
        cmy2 = jnp.where(valid, comp[2:3, :], sent_row)
        cmx2 = jnp.where(valid, comp[3:4, :], sent_row)
        cmar = jnp.where(valid, comp[4:5, :], one_row)

        rly1 = pltpu.roll(cmy1, c, 1)
        rlx1 = pltpu.roll(cmx1, c, 1)
        rly2 = pltpu.roll(cmy2, c, 1)
        rlx2 = pltpu.roll(cmx2, c, 1)
        rlar = pltpu.roll(cmar, c, 1)
        in_hi = lane >= c
        mgy1 = jnp.where(in_hi, rly1, gy1)
        mgx1 = jnp.where(in_hi, rlx1, gx1)
        mgy2 = jnp.where(in_hi, rly2, gy2)
        mgx2 = jnp.where(in_hi, rlx2, gx2)
        mgar = jnp.where(in_hi, rlar, gar)

        flush = (c + k) >= _L
        c_new = jnp.where(flush, c + k - _L, c + k)

        @pl.when(flush)
        def _():
            sy1[pl.ds(nfull, 1), :] = mgy1
            sx1[pl.ds(nfull, 1), :] = mgx1
            sy2[pl.ds(nfull, 1), :] = mgy2
            sx2[pl.ds(nfull, 1), :] = mgx2
            sar[pl.ds(nfull, 1), :] = mgar

        nfull_new = nfull + jnp.where(flush, 1, 0)
        in_lo = lane < c_new
        ny1 = jnp.where(flush, jnp.where(in_lo, rly1, sent_row), mgy1)
        nx1 = jnp.where(flush, jnp.where(in_lo, rlx1, sent_row), mgx1)
        ny2 = jnp.where(flush, jnp.where(in_lo, rly2, sent_row), mgy2)
        nx2 = jnp.where(flush, jnp.where(in_lo, rlx2, sent_row), mgx2)
        nar = jnp.where(flush, jnp.where(in_lo, rlar, one_row), mgar)

        return (nfull_new, c_new, ny1, nx1, ny2, nx2, nar)

    init = (jnp.int32(0), jnp.int32(0), sent_row, sent_row, sent_row,
            sent_row, one_row)
    lax.fori_loop(0, nrows, block_step, init)


def kernel(anchors, deltas, scores):
    n = scores.shape[0]
    nrows = (n + _L - 1) // _L
    pad = nrows * _L - n

    order = jnp.argsort(-scores)
    sa = jnp.pad(anchors[order], ((0, pad), (0, 0)))
    sd = jnp.pad(deltas[order], ((0, pad), (0, 0)))
    ss = scores[order]

    sa4 = sa.T.reshape(4, nrows, _L)
    sd4 = sd.T.reshape(4, nrows, _L)

    nsrv = ((nrows + 8) // 8) * 8
    boxes4, keep = pl.pallas_call(
        functools.partial(_nms_body, nrows=nrows, nsrv=nsrv),
        out_shape=[
            jax.ShapeDtypeStruct((4, nrows, _L), jnp.float32),
            jax.ShapeDtypeStruct((nrows, _L), jnp.bool_),
        ],
        scratch_shapes=(
            [pltpu.VMEM((nrows, _L), jnp.float32)] * 5
            + [pltpu.VMEM((nsrv, _L), jnp.float32)] * 5
            + [pltpu.VMEM((_L, _L), jnp.int32)]
        ),
    )(sa4, sd4)

    boxes_s = boxes4.reshape(4, nrows * _L)[:, :n].T
    keep_s = keep.reshape(nrows * _L)[:n]
    dets_sorted = jnp.concatenate([boxes_s, ss[:, None]], axis=1)
    dets_sorted = dets_sorted * keep_s[:, None].astype(jnp.float32)
    return jnp.zeros((n, 5), jnp.float32).at[order].add(dets_sorted)
